# baseline (speedup 1.0000x reference)
import jax
import jax.numpy as jnp
from jax.experimental import pallas as pl
from jax.experimental.pallas import tpu as pltpu

_N = 1200
_D = 512
_H = 42
_C = 21


def _gcn_body(x1_hbm, br_hbm, wc1_hbm, bc1_ref, wc2_hbm, bc2_ref,
              wd1_hbm, bd1_ref, wd2_hbm, bd2_ref, cls_ref, det_ref,
              x1_v, br_v, wc1_v, wc2_v, wd1_v, wd2_v, sem):
    cps = (pltpu.make_async_copy(br_hbm, br_v, sem.at[0]),
           pltpu.make_async_copy(wc1_hbm, wc1_v, sem.at[1]),
           pltpu.make_async_copy(wc2_hbm, wc2_v, sem.at[2]),
           pltpu.make_async_copy(wd1_hbm, wd1_v, sem.at[3]),
           pltpu.make_async_copy(wd2_hbm, wd2_v, sem.at[4]),
           pltpu.make_async_copy(x1_hbm, x1_v, sem.at[5]))
    for c in cps:
        c.start()
    for c in cps:
        c.wait()

    bt = jnp.transpose(br_v[:])
    cx1 = br_v[0:1, :]; cy1 = br_v[1:2, :]
    cx2 = br_v[2:3, :]; cy2 = br_v[3:4, :]
    rx1 = bt[:, 0:1]; ry1 = bt[:, 1:2]
    rx2 = bt[:, 2:3]; ry2 = bt[:, 3:4]
    area_r = (rx2 - rx1) * (ry2 - ry1)
    area_c = (cx2 - cx1) * (cy2 - cy1)
    iw = jnp.maximum(jnp.minimum(rx2, cx2) - jnp.maximum(rx1, cx1), 0.0)
    ih = jnp.maximum(jnp.minimum(ry2, cy2) - jnp.maximum(ry1, cy1), 0.0)
    inter = iw * ih
    ma = (3.0 * inter >= area_r + area_c).astype(jnp.float32)

    x1 = x1_v[:]
    nrm = jnp.sqrt(jnp.sum(x1 * x1, axis=1, keepdims=True))
    xh = x1 / jnp.maximum(nrm, 1e-12)
    sim = jax.lax.dot_general(xh, xh, (((1,), (1,)), ((), ())),
                              preferred_element_type=jnp.float32)
    ms = (sim >= 0.5).astype(jnp.float32)

    rowsum = jnp.sum(x1, axis=1, keepdims=True)
    rinv = jnp.where(jnp.abs(rowsum) > 1e-12, 1.0 / rowsum, 0.0)
    xn = x1 * rinv

    def branch(m, w1t, b1, w2t, b2):
        h1t = jax.lax.dot_general(w1t, xn, (((1,), (1,)), ((), ())),
                                  preferred_element_type=jnp.float32)
        t1t = jnp.dot(h1t, m, preferred_element_type=jnp.float32)
        z1t = jax.nn.relu(t1t + b1[:, None])
        h2t = jnp.dot(w2t, z1t, preferred_element_type=jnp.float32)
        return jnp.dot(h2t, m, preferred_element_type=jnp.float32) + b2[:, None]

    clst = branch(ma, wc1_v[:], bc1_ref[:], wc2_v[:], bc2_ref[:])
    dett = branch(ms, wd1_v[:], bd1_ref[:], wd2_v[:], bd2_ref[:])

    clst = clst - jnp.max(clst, axis=0, keepdims=True)
    ec = jnp.exp(clst)
    cls_ref[:] = ec / jnp.sum(ec, axis=0, keepdims=True)

    dett = dett - jnp.max(dett, axis=1, keepdims=True)
    ed = jnp.exp(dett)
    det_ref[:] = ed / jnp.sum(ed, axis=1, keepdims=True)


_SCRATCH = (
    pltpu.MemorySpace.VMEM((_N, _D), jnp.float32),
    pltpu.MemorySpace.VMEM((4, _N), jnp.float32),
    pltpu.MemorySpace.VMEM((_H, _D), jnp.float32),
    pltpu.MemorySpace.VMEM((_C, _H), jnp.float32),
    pltpu.MemorySpace.VMEM((_H, _D), jnp.float32),
    pltpu.MemorySpace.VMEM((_C, _H), jnp.float32),
    pltpu.SemaphoreType.DMA((6,)),
)

_HBM_SPEC = pl.BlockSpec(memory_space=pltpu.MemorySpace.HBM)
_VMEM_SPEC = pl.BlockSpec(memory_space=pltpu.MemorySpace.VMEM)
_IN_SPECS = [_HBM_SPEC, _HBM_SPEC, _HBM_SPEC, _VMEM_SPEC, _HBM_SPEC,
             _VMEM_SPEC, _HBM_SPEC, _VMEM_SPEC, _HBM_SPEC, _VMEM_SPEC]


@jax.jit
def kernel(x1, x2, proposal_boxes, Wc1, bc1, Wc2, bc2, Wd1, bd1, Wd2, bd2):
    del x2
    _hbm = lambda a: pltpu.with_memory_space_constraint(a, pltpu.MemorySpace.HBM)
    clst, dett = pl.pallas_call(
        _gcn_body,
        in_specs=_IN_SPECS,
        out_shape=(jax.ShapeDtypeStruct((_C, _N), jnp.float32),
                   jax.ShapeDtypeStruct((_C, _N), jnp.float32)),
        scratch_shapes=_SCRATCH,
        compiler_params=pltpu.CompilerParams(skip_device_barrier=True),
    )(_hbm(x1), _hbm(proposal_boxes.T), _hbm(Wc1.T), bc1, _hbm(Wc2.T), bc2,
      _hbm(Wd1.T), bd1, _hbm(Wd2.T), bd2)
    return clst.T, dett.T
